# baseline (speedup 1.0000x reference)
import functools
import math

import jax
import jax.numpy as jnp
from jax.experimental import pallas as pl
from jax.experimental.pallas import tpu as pltpu

_THRESHOLD = 0.1
_BLK = 256


def _mask_kernel(ego_ref, cavr_ref, cavc_ref, mask_ref, f_sc, *, s_total, k):
    ego_b = ego_ref[0, 0].astype(jnp.bfloat16)
    cavr_b = cavr_ref[0, 0].astype(jnp.bfloat16)
    cavc_b = cavc_ref[0, 0].astype(jnp.bfloat16)
    sqrt_c = jnp.float32(math.sqrt(2.0))

    for r in range(s_total // _BLK):
        p0 = r * _BLK
        q_b = ego_b[:, p0:p0 + _BLK]
        sim = jax.lax.dot_general(
            cavc_b, q_b, (((1,), (0,)), ((), ())),
            preferred_element_type=jnp.float32)
        sim = sim / sqrt_c
        m = jnp.max(sim, axis=0, keepdims=True)
        e = jnp.exp(sim - m)
        den = jnp.sum(e, axis=0, keepdims=True)
        attn_b = (e / den).astype(jnp.bfloat16)
        y = jax.lax.dot_general(
            cavr_b, attn_b, (((1,), (0,)), ((), ())),
            preferred_element_type=jnp.float32)
        z = jnp.max(y, axis=0, keepdims=True)
        f_sc[0:1, p0:p0 + _BLK] = jax.nn.sigmoid(z)

    f = f_sc[0:1, :]
    keys = jax.lax.bitcast_convert_type(f, jnp.int32)
    t = jnp.int32(0)
    for bit in range(30, -1, -1):
        cand = t | jnp.int32(1 << bit)
        cnt = jnp.sum((keys >= cand).astype(jnp.int32))
        t = jnp.where(cnt >= k, cand, t)
    mask_ref[0, 0] = (keys >= t).astype(jnp.float32)


def _compute_masks(psm):
    B, L, C2, H, W = psm.shape
    S = H * W
    psm_r = psm.reshape(B, L, C2, S)
    psm_c = jnp.swapaxes(psm_r, -1, -2)
    k = max(1, int(S * _THRESHOLD))
    kern = functools.partial(_mask_kernel, s_total=S, k=k)
    return pl.pallas_call(
        kern,
        grid=(B, L - 1),
        in_specs=[
            pl.BlockSpec((1, 1, C2, S), lambda b, j: (b, 0, 0, 0)),
            pl.BlockSpec((1, 1, C2, S), lambda b, j: (b, j + 1, 0, 0)),
            pl.BlockSpec((1, 1, S, C2), lambda b, j: (b, j + 1, 0, 0)),
        ],
        out_specs=pl.BlockSpec((1, 1, 1, S), lambda b, j: (b, j, 0, 0)),
        out_shape=jax.ShapeDtypeStruct((B, L - 1, 1, S), jnp.float32),
        scratch_shapes=[pltpu.VMEM((1, S), jnp.float32)],
        compiler_params=pltpu.CompilerParams(
            dimension_semantics=("parallel", "parallel")),
    )(psm_r, psm_r, psm_c)


def _apply_kernel(x_ref, m_ref, o_ref):
    o_ref[...] = x_ref[...] * m_ref[...]


def kernel(x, psm, mask):
    B, L, C, H, W = x.shape
    S = H * W
    masks = _compute_masks(psm).reshape(B, L - 1, S)
    gate = (mask[:, 1:] != 0).astype(jnp.float32)[:, :, None]
    m_full = jnp.concatenate(
        [jnp.ones((B, 1, S), jnp.float32), masks * gate], axis=1
    ).reshape(B, L, 1, S)
    xr = x.reshape(B, L, C, S)
    cb = 32
    out = pl.pallas_call(
        _apply_kernel,
        grid=(B, L, C // cb),
        in_specs=[
            pl.BlockSpec((1, 1, cb, S), lambda b, l, c: (b, l, c, 0)),
            pl.BlockSpec((1, 1, 1, S), lambda b, l, c: (b, l, 0, 0)),
        ],
        out_specs=pl.BlockSpec((1, 1, cb, S), lambda b, l, c: (b, l, c, 0)),
        out_shape=jax.ShapeDtypeStruct((B, L, C, S), x.dtype),
        compiler_params=pltpu.CompilerParams(
            dimension_semantics=("parallel", "parallel", "parallel")),
    )(xr, m_full)
    return out.reshape(B, L, C, H, W)


# speedup vs baseline: 1.0966x; 1.0966x over previous
import functools
import math

import jax
import jax.numpy as jnp
from jax.experimental import pallas as pl
from jax.experimental.pallas import tpu as pltpu

_THRESHOLD = 0.1
_BLK = 128


def _mask_kernel(ego_ref, cavr_ref, cavc_ref, mask_ref, f_sc, *, s_total, k):
    def _bf(v):
        return v.astype(jnp.bfloat16).astype(jnp.float32)

    ego = ego_ref[0, 0]
    del cavr_ref
    cavc = cavc_ref[0, 0]
    u_bf = _bf(cavc[:, 0:1])
    v_bf = _bf(cavc[:, 1:2])
    sqrt_c = jnp.float32(math.sqrt(2.0))

    for r in range(s_total // _BLK):
        p0 = r * _BLK
        a = _bf(ego[0:1, p0:p0 + _BLK])
        b = _bf(ego[1:2, p0:p0 + _BLK])
        sim = (u_bf * a + v_bf * b) / sqrt_c
        m = jnp.max(sim, axis=0, keepdims=True)
        e = jnp.exp(sim - m)
        den = jnp.sum(e, axis=0, keepdims=True)
        attn = _bf(e / den)
        y0 = jnp.sum(attn * u_bf, axis=0, keepdims=True)
        y1 = jnp.sum(attn * v_bf, axis=0, keepdims=True)
        z = jnp.maximum(y0, y1)
        f_sc[0:1, p0:p0 + _BLK] = jax.nn.sigmoid(z)

    f = f_sc[0:1, :]
    keys = jax.lax.bitcast_convert_type(f, jnp.int32)
    t = jnp.int32(0)
    for bit in range(30, -1, -1):
        cand = t | jnp.int32(1 << bit)
        cnt = jnp.sum((keys >= cand).astype(jnp.int32))
        t = jnp.where(cnt >= k, cand, t)
    mask_ref[0, 0] = (keys >= t).astype(jnp.float32)


def _compute_masks(psm):
    B, L, C2, H, W = psm.shape
    S = H * W
    psm_r = psm.reshape(B, L, C2, S)
    psm_c = jnp.swapaxes(psm_r, -1, -2)
    k = max(1, int(S * _THRESHOLD))
    kern = functools.partial(_mask_kernel, s_total=S, k=k)
    return pl.pallas_call(
        kern,
        grid=(B, L - 1),
        in_specs=[
            pl.BlockSpec((1, 1, C2, S), lambda b, j: (b, 0, 0, 0)),
            pl.BlockSpec((1, 1, C2, S), lambda b, j: (b, j + 1, 0, 0)),
            pl.BlockSpec((1, 1, S, C2), lambda b, j: (b, j + 1, 0, 0)),
        ],
        out_specs=pl.BlockSpec((1, 1, 1, S), lambda b, j: (b, j, 0, 0)),
        out_shape=jax.ShapeDtypeStruct((B, L - 1, 1, S), jnp.float32),
        scratch_shapes=[pltpu.VMEM((1, S), jnp.float32)],
        compiler_params=pltpu.CompilerParams(
            dimension_semantics=("parallel", "parallel")),
    )(psm_r, psm_r, psm_c)


def _apply_kernel(x_ref, m_ref, o_ref):
    o_ref[...] = x_ref[...] * m_ref[...]


def kernel(x, psm, mask):
    B, L, C, H, W = x.shape
    S = H * W
    masks = _compute_masks(psm).reshape(B, L - 1, S)
    gate = (mask[:, 1:] != 0).astype(jnp.float32)[:, :, None]
    m_full = jnp.concatenate(
        [jnp.ones((B, 1, S), jnp.float32), masks * gate], axis=1
    ).reshape(B, L, 1, S)
    xr = x.reshape(B, L, C, S)
    cb = 32
    out = pl.pallas_call(
        _apply_kernel,
        grid=(B, L, C // cb),
        in_specs=[
            pl.BlockSpec((1, 1, cb, S), lambda b, l, c: (b, l, c, 0)),
            pl.BlockSpec((1, 1, 1, S), lambda b, l, c: (b, l, 0, 0)),
        ],
        out_specs=pl.BlockSpec((1, 1, cb, S), lambda b, l, c: (b, l, c, 0)),
        out_shape=jax.ShapeDtypeStruct((B, L, C, S), x.dtype),
        compiler_params=pltpu.CompilerParams(
            dimension_semantics=("parallel", "parallel", "parallel")),
    )(xr, m_full)
    return out.reshape(B, L, C, H, W)


# speedup vs baseline: 1.1558x; 1.0540x over previous
import functools
import math

import jax
import jax.numpy as jnp
from jax.experimental import pallas as pl
from jax.experimental.pallas import tpu as pltpu

_THRESHOLD = 0.1
_BLK = 128


def _mask_kernel(ego_ref, cav_ref, f_ref, *, s_total):
    ego_b16 = ego_ref[0, 0].astype(jnp.bfloat16)
    cav = cav_ref[0, 0]
    u_row = cav[0:1, :].astype(jnp.bfloat16).astype(jnp.float32)
    v_row = cav[1:2, :].astype(jnp.bfloat16).astype(jnp.float32)
    sqrt_c = jnp.float32(math.sqrt(2.0))
    rows = jax.lax.broadcasted_iota(jnp.int32, (_BLK, _BLK), 0)
    cols = jax.lax.broadcasted_iota(jnp.int32, (_BLK, _BLK), 1)
    eye_b16 = (rows == cols).astype(jnp.bfloat16)

    for r in range(s_total // _BLK):
        p0 = r * _BLK
        ab_cols = jax.lax.dot_general(
            eye_b16, ego_b16[:, p0:p0 + _BLK], (((1,), (1,)), ((), ())),
            preferred_element_type=jnp.float32)
        a_col = ab_cols[:, 0:1]
        b_col = ab_cols[:, 1:2]
        sim = (a_col * u_row + b_col * v_row) / sqrt_c
        m = jnp.max(sim, axis=1, keepdims=True)
        e = jnp.exp(sim - m)
        den = jnp.sum(e, axis=1, keepdims=True)
        attn = (e / den).astype(jnp.bfloat16).astype(jnp.float32)
        y0 = jnp.sum(attn * u_row, axis=1, keepdims=True)
        y1 = jnp.sum(attn * v_row, axis=1, keepdims=True)
        z = jnp.maximum(y0, y1)
        f_ref[0, 0, p0:p0 + _BLK, 0:1] = jax.nn.sigmoid(z)


def _threshold_kernel(f_ref, gate_ref, mask_ref, *, k):
    f = f_ref[0, 0]
    keys = jax.lax.bitcast_convert_type(f, jnp.int32)
    t = jnp.int32(0)
    for bit in range(30, -1, -1):
        cand = t | jnp.int32(1 << bit)
        cnt = jnp.sum((keys >= cand).astype(jnp.int32))
        t = jnp.where(cnt >= k, cand, t)
    g = gate_ref[0, 0, 0, 0]
    mask_ref[0, 0] = (keys >= t).astype(jnp.float32) * g


def _compute_masks(psm, gate):
    B, L, C2, H, W = psm.shape
    S = H * W
    psm_r = psm.reshape(B, L, C2, S)
    k = max(1, int(S * _THRESHOLD))
    kern = functools.partial(_mask_kernel, s_total=S)
    f = pl.pallas_call(
        kern,
        grid=(B, L - 1),
        in_specs=[
            pl.BlockSpec((1, 1, C2, S), lambda b, j: (b, 0, 0, 0)),
            pl.BlockSpec((1, 1, C2, S), lambda b, j: (b, j + 1, 0, 0)),
        ],
        out_specs=pl.BlockSpec((1, 1, S, 1), lambda b, j: (b, j, 0, 0)),
        out_shape=jax.ShapeDtypeStruct((B, L - 1, S, 1), jnp.float32),
        compiler_params=pltpu.CompilerParams(
            dimension_semantics=("parallel", "parallel")),
    )(psm_r, psm_r)
    return pl.pallas_call(
        functools.partial(_threshold_kernel, k=k),
        grid=(B, L - 1),
        in_specs=[
            pl.BlockSpec((1, 1, 1, S), lambda b, j: (b, j, 0, 0)),
            pl.BlockSpec((1, 1, 1, 1), lambda b, j: (b, j, 0, 0)),
        ],
        out_specs=pl.BlockSpec((1, 1, 1, S), lambda b, j: (b, j, 0, 0)),
        out_shape=jax.ShapeDtypeStruct((B, L - 1, 1, S), jnp.float32),
        compiler_params=pltpu.CompilerParams(
            dimension_semantics=("parallel", "parallel")),
    )(f.reshape(B, L - 1, 1, S), gate)


def _apply_kernel(x_ref, m_ref, o_ref):
    @pl.when(pl.program_id(1) == 0)
    def _copy():
        o_ref[...] = x_ref[...]

    @pl.when(pl.program_id(1) != 0)
    def _mask():
        o_ref[...] = x_ref[...] * m_ref[...]


def kernel(x, psm, mask):
    B, L, C, H, W = x.shape
    S = H * W
    gate = (mask[:, 1:] != 0).astype(jnp.float32).reshape(B, L - 1, 1, 1)
    masks = _compute_masks(psm, gate)
    xr = x.reshape(B, L, C, S)
    cb = 64
    out = pl.pallas_call(
        _apply_kernel,
        grid=(B, L, C // cb),
        in_specs=[
            pl.BlockSpec((1, 1, cb, S), lambda b, l, c: (b, l, c, 0)),
            pl.BlockSpec((1, 1, 1, S),
                         lambda b, l, c: (b, jnp.maximum(l - 1, 0), 0, 0)),
        ],
        out_specs=pl.BlockSpec((1, 1, cb, S), lambda b, l, c: (b, l, c, 0)),
        out_shape=jax.ShapeDtypeStruct((B, L, C, S), x.dtype),
        compiler_params=pltpu.CompilerParams(
            dimension_semantics=("parallel", "parallel", "parallel")),
    )(xr, masks)
    return out.reshape(B, L, C, H, W)


# speedup vs baseline: 1.4067x; 1.2171x over previous
import functools
import math

import jax
import jax.numpy as jnp
from jax.experimental import pallas as pl
from jax.experimental.pallas import tpu as pltpu

_THRESHOLD = 0.1
_BLK = 128


def _mask_kernel(ego_ref, cav_ref, f_ref, *, s_total):
    ego_b16 = ego_ref[0, 0].astype(jnp.bfloat16)
    cav_b16 = cav_ref[0, 0].astype(jnp.bfloat16)
    sqrt_c = jnp.float32(math.sqrt(2.0))
    rows = jax.lax.broadcasted_iota(jnp.int32, (_BLK, _BLK), 0)
    cols = jax.lax.broadcasted_iota(jnp.int32, (_BLK, _BLK), 1)
    eye_b16 = (rows == cols).astype(jnp.bfloat16)

    for r in range(s_total // _BLK):
        p0 = r * _BLK
        ab_cols = jax.lax.dot_general(
            eye_b16, ego_b16[:, p0:p0 + _BLK], (((1,), (1,)), ((), ())),
            preferred_element_type=jnp.float32)
        sim = jax.lax.dot_general(
            ab_cols.astype(jnp.bfloat16), cav_b16, (((1,), (0,)), ((), ())),
            preferred_element_type=jnp.float32) / sqrt_c
        m = jnp.max(sim, axis=1, keepdims=True)
        e = jnp.exp(sim - m)
        den = jnp.sum(e, axis=1, keepdims=True)
        attn_b16 = (e / den).astype(jnp.bfloat16)
        y = jax.lax.dot_general(
            attn_b16, cav_b16, (((1,), (1,)), ((), ())),
            preferred_element_type=jnp.float32)
        z = jnp.max(y, axis=1, keepdims=True)
        f_ref[0, 0, p0:p0 + _BLK, 0:1] = jax.nn.sigmoid(z)


def _threshold_kernel(f_ref, gate_ref, mask_ref, *, k):
    f = f_ref[0, 0]
    keys = jax.lax.bitcast_convert_type(f, jnp.int32)
    t = jnp.int32(0)
    for bit in range(30, -1, -1):
        cand = t | jnp.int32(1 << bit)
        cnt = jnp.sum((keys >= cand).astype(jnp.int32))
        t = jnp.where(cnt >= k, cand, t)
    g = gate_ref[0, 0, 0, 0]
    mask_ref[0, 0] = (keys >= t).astype(jnp.float32) * g


def _compute_masks(psm, gate):
    B, L, C2, H, W = psm.shape
    S = H * W
    psm_r = psm.reshape(B, L, C2, S)
    k = max(1, int(S * _THRESHOLD))
    kern = functools.partial(_mask_kernel, s_total=S)
    f = pl.pallas_call(
        kern,
        grid=(B, L - 1),
        in_specs=[
            pl.BlockSpec((1, 1, C2, S), lambda b, j: (b, 0, 0, 0)),
            pl.BlockSpec((1, 1, C2, S), lambda b, j: (b, j + 1, 0, 0)),
        ],
        out_specs=pl.BlockSpec((1, 1, S, 1), lambda b, j: (b, j, 0, 0)),
        out_shape=jax.ShapeDtypeStruct((B, L - 1, S, 1), jnp.float32),
        compiler_params=pltpu.CompilerParams(
            dimension_semantics=("parallel", "parallel")),
    )(psm_r, psm_r)
    return pl.pallas_call(
        functools.partial(_threshold_kernel, k=k),
        grid=(B, L - 1),
        in_specs=[
            pl.BlockSpec((1, 1, 1, S), lambda b, j: (b, j, 0, 0)),
            pl.BlockSpec((1, 1, 1, 1), lambda b, j: (b, j, 0, 0)),
        ],
        out_specs=pl.BlockSpec((1, 1, 1, S), lambda b, j: (b, j, 0, 0)),
        out_shape=jax.ShapeDtypeStruct((B, L - 1, 1, S), jnp.float32),
        compiler_params=pltpu.CompilerParams(
            dimension_semantics=("parallel", "parallel")),
    )(f.reshape(B, L - 1, 1, S), gate)


def _apply_kernel(x_ref, m_ref, o_ref):
    @pl.when(pl.program_id(1) == 0)
    def _copy():
        o_ref[...] = x_ref[...]

    @pl.when(pl.program_id(1) != 0)
    def _mask():
        o_ref[...] = x_ref[...] * m_ref[...]


def kernel(x, psm, mask):
    B, L, C, H, W = x.shape
    S = H * W
    gate = (mask[:, 1:] != 0).astype(jnp.float32).reshape(B, L - 1, 1, 1)
    masks = _compute_masks(psm, gate)
    xr = x.reshape(B, L, C, S)
    cb = 64
    out = pl.pallas_call(
        _apply_kernel,
        grid=(B, L, C // cb),
        in_specs=[
            pl.BlockSpec((1, 1, cb, S), lambda b, l, c: (b, l, c, 0)),
            pl.BlockSpec((1, 1, 1, S),
                         lambda b, l, c: (b, jnp.maximum(l - 1, 0), 0, 0)),
        ],
        out_specs=pl.BlockSpec((1, 1, cb, S), lambda b, l, c: (b, l, c, 0)),
        out_shape=jax.ShapeDtypeStruct((B, L, C, S), x.dtype),
        compiler_params=pltpu.CompilerParams(
            dimension_semantics=("parallel", "parallel", "parallel")),
    )(xr, masks)
    return out.reshape(B, L, C, H, W)


# speedup vs baseline: 1.5507x; 1.1023x over previous
import functools
import math

import jax
import jax.numpy as jnp
from jax.experimental import pallas as pl
from jax.experimental.pallas import tpu as pltpu

_THRESHOLD = 0.1
_BLK = 256


def _mask_kernel(ego_ref, cav_ref, f_ref, *, s_total):
    ego_b16 = ego_ref[0, 0].astype(jnp.bfloat16)
    cav_b16 = cav_ref[0, 0].astype(jnp.bfloat16)
    sqrt_c = jnp.float32(math.sqrt(2.0))
    rows = jax.lax.broadcasted_iota(jnp.int32, (_BLK, _BLK), 0)
    cols = jax.lax.broadcasted_iota(jnp.int32, (_BLK, _BLK), 1)
    eye_b16 = (rows == cols).astype(jnp.bfloat16)

    for r in range(s_total // _BLK):
        p0 = r * _BLK
        ab_cols = jax.lax.dot_general(
            eye_b16, ego_b16[:, p0:p0 + _BLK], (((1,), (1,)), ((), ())),
            preferred_element_type=jnp.float32)
        sim = jax.lax.dot_general(
            ab_cols.astype(jnp.bfloat16), cav_b16, (((1,), (0,)), ((), ())),
            preferred_element_type=jnp.float32) / sqrt_c
        m = jnp.max(sim, axis=1, keepdims=True)
        e = jnp.exp(sim - m)
        den = jnp.sum(e, axis=1, keepdims=True)
        attn_b16 = (e / den).astype(jnp.bfloat16)
        y = jax.lax.dot_general(
            attn_b16, cav_b16, (((1,), (1,)), ((), ())),
            preferred_element_type=jnp.float32)
        z = jnp.max(y, axis=1, keepdims=True)
        f_ref[0, 0, p0:p0 + _BLK, 0:1] = jax.nn.sigmoid(z)


def _threshold_kernel(f_ref, gate_ref, mask_ref, *, k):
    f = f_ref[0, 0]
    keys = jax.lax.bitcast_convert_type(f, jnp.int32)
    t = jnp.int32(0)
    for bit in range(30, -1, -1):
        cand = t | jnp.int32(1 << bit)
        cnt = jnp.sum((keys >= cand).astype(jnp.int32))
        t = jnp.where(cnt >= k, cand, t)
    g = gate_ref[0, 0, 0, 0]
    mask_ref[0, 0] = (keys >= t).astype(jnp.float32) * g


def _compute_masks(psm, gate):
    B, L, C2, H, W = psm.shape
    S = H * W
    psm_r = psm.reshape(B, L, C2, S)
    k = max(1, int(S * _THRESHOLD))
    kern = functools.partial(_mask_kernel, s_total=S)
    f = pl.pallas_call(
        kern,
        grid=(B, L - 1),
        in_specs=[
            pl.BlockSpec((1, 1, C2, S), lambda b, j: (b, 0, 0, 0)),
            pl.BlockSpec((1, 1, C2, S), lambda b, j: (b, j + 1, 0, 0)),
        ],
        out_specs=pl.BlockSpec((1, 1, S, 1), lambda b, j: (b, j, 0, 0)),
        out_shape=jax.ShapeDtypeStruct((B, L - 1, S, 1), jnp.float32),
        compiler_params=pltpu.CompilerParams(
            dimension_semantics=("parallel", "parallel")),
    )(psm_r, psm_r)
    return pl.pallas_call(
        functools.partial(_threshold_kernel, k=k),
        grid=(B, L - 1),
        in_specs=[
            pl.BlockSpec((1, 1, 1, S), lambda b, j: (b, j, 0, 0)),
            pl.BlockSpec((1, 1, 1, 1), lambda b, j: (b, j, 0, 0)),
        ],
        out_specs=pl.BlockSpec((1, 1, 1, S), lambda b, j: (b, j, 0, 0)),
        out_shape=jax.ShapeDtypeStruct((B, L - 1, 1, S), jnp.float32),
        compiler_params=pltpu.CompilerParams(
            dimension_semantics=("parallel", "parallel")),
    )(f.reshape(B, L - 1, 1, S), gate)


def _apply_kernel(x_ref, m_ref, o_ref):
    @pl.when(pl.program_id(1) == 0)
    def _copy():
        o_ref[...] = x_ref[...]

    @pl.when(pl.program_id(1) != 0)
    def _mask():
        o_ref[...] = x_ref[...] * m_ref[...]


def kernel(x, psm, mask):
    B, L, C, H, W = x.shape
    S = H * W
    gate = (mask[:, 1:] != 0).astype(jnp.float32).reshape(B, L - 1, 1, 1)
    masks = _compute_masks(psm, gate)
    xr = x.reshape(B, L, C, S)
    cb = 64
    out = pl.pallas_call(
        _apply_kernel,
        grid=(B, L, C // cb),
        in_specs=[
            pl.BlockSpec((1, 1, cb, S), lambda b, l, c: (b, l, c, 0)),
            pl.BlockSpec((1, 1, 1, S),
                         lambda b, l, c: (b, jnp.maximum(l - 1, 0), 0, 0)),
        ],
        out_specs=pl.BlockSpec((1, 1, cb, S), lambda b, l, c: (b, l, c, 0)),
        out_shape=jax.ShapeDtypeStruct((B, L, C, S), x.dtype),
        compiler_params=pltpu.CompilerParams(
            dimension_semantics=("parallel", "parallel", "parallel")),
    )(xr, masks)
    return out.reshape(B, L, C, H, W)
